# Initial kernel scaffold; baseline (speedup 1.0000x reference)
import jax
import jax.numpy as jnp
from jax import lax
from jax.experimental import pallas as pl
from jax.experimental.pallas import tpu as pltpu
from jax.experimental.pallas import tpu_sc as plsc

NUSERS = 25000
NNODES = 50000
ROWB = 256
NBLK = 196
NP = NBLK * ROWB
EMB = 64
HID = 128
NE = 800000
NC, NS = 2, 16
BT = 128
NEPAD = NC * NS * BT * 196
NER = NEPAD // BT
STRIPE = NP // NS
BSZ = 16384

_MESH = plsc.VectorSubcoreMesh(
    core_axis_name="c", subcore_axis_name="s", num_cores=NC, num_subcores=NS)


def _deg_body(dst_hbm, zero16_hbm, ones_hbm, deg0_hbm, deg1_hbm,
              idx_all, ones_v, acc, sem):
    c = lax.axis_index("c")
    s = lax.axis_index("s")
    r0 = s * STRIPE
    pltpu.sync_copy(zero16_hbm.at[pl.ds(r0, STRIPE)], acc.at[pl.ds(r0, STRIPE)])
    pltpu.sync_copy(ones_hbm, ones_v)
    rbase = c * (NER // 2) + s * (NER // 2 // NS)
    pltpu.sync_copy(dst_hbm.at[pl.ds(rbase, 196)], idx_all)
    plsc.subcore_barrier()

    def body(t, _):
        pltpu.sync_copy(ones_v, acc.at[idx_all.at[t]], add=True)
        return ()

    lax.fori_loop(0, 196, body, ())
    plsc.subcore_barrier()

    @pl.when(c == 0)
    def _():
        pltpu.sync_copy(acc.at[pl.ds(r0, STRIPE)], deg0_hbm.at[pl.ds(r0, STRIPE)])

    @pl.when(c == 1)
    def _():
        pltpu.sync_copy(acc.at[pl.ds(r0, STRIPE)], deg1_hbm.at[pl.ds(r0, STRIPE)])


_deg_call = pl.kernel(
    _deg_body,
    out_type=(jax.ShapeDtypeStruct((NP, 16), jnp.float32),
              jax.ShapeDtypeStruct((NP, 16), jnp.float32)),
    mesh=_MESH,
    scratch_types=[
        pltpu.VMEM((196, BT), jnp.int32),
        pltpu.VMEM((BT, 16), jnp.float32),
        pltpu.VMEM_SHARED((NP, 16), jnp.float32),
        pltpu.SemaphoreType.DMA,
    ],
)


def _spmm_body(src_hbm, dst_hbm, ga_hbm, gb_hbm, zero32_hbm,
               sa_hbm, sb_hbm,
               idxs_all, idxd_all, rows, acc, sem):
    c = lax.axis_index("c")
    s = lax.axis_index("s")
    r0 = s * STRIPE
    nb = NER // NS
    rbase = s * nb
    pltpu.sync_copy(zero32_hbm.at[pl.ds(r0, STRIPE)], acc.at[pl.ds(r0, STRIPE)])
    pltpu.sync_copy(src_hbm.at[pl.ds(rbase, nb)], idxs_all)
    pltpu.sync_copy(dst_hbm.at[pl.ds(rbase, nb)], idxd_all)
    plsc.subcore_barrier()

    def run(g_hbm, s_out):
        def body(t, _):
            pltpu.async_copy(g_hbm.at[idxs_all.at[t]], rows, sem).wait()
            pltpu.sync_copy(rows, acc.at[idxd_all.at[t]], add=True)
            return ()

        lax.fori_loop(0, nb, body, ())
        plsc.subcore_barrier()
        pltpu.sync_copy(acc.at[pl.ds(r0, STRIPE)], s_out.at[pl.ds(r0, STRIPE)])

    @pl.when(c == 0)
    def _():
        run(ga_hbm, sa_hbm)

    @pl.when(c == 1)
    def _():
        run(gb_hbm, sb_hbm)


_spmm_call = pl.kernel(
    _spmm_body,
    out_type=(jax.ShapeDtypeStruct((NP, 32), jnp.float32),
              jax.ShapeDtypeStruct((NP, 32), jnp.float32)),
    mesh=_MESH,
    scratch_types=[
        pltpu.VMEM((NER // NS, BT), jnp.int32),
        pltpu.VMEM((NER // NS, BT), jnp.int32),
        pltpu.VMEM((BT, 32), jnp.float32),
        pltpu.VMEM_SHARED((NP, 32), jnp.float32),
        pltpu.SemaphoreType.DMA,
    ],
)


def _emb_body(h2_hbm, uidx_hbm, iidx_hbm, ue_hbm, ie_hbm, idx_v, rows, sem):
    c = lax.axis_index("c")
    s = lax.axis_index("s")
    wid = c * NS + s
    base = wid * (BSZ // (NC * NS))

    def make(src_idx_hbm, out_hbm):
        def body(t, _):
            off = base + t * BT
            pltpu.sync_copy(src_idx_hbm.at[pl.ds(off, BT)], idx_v)
            pltpu.async_copy(h2_hbm.at[idx_v], rows, sem).wait()
            pltpu.sync_copy(rows, out_hbm.at[pl.ds(off, BT)])
            return ()
        return body

    lax.fori_loop(0, 4, make(uidx_hbm, ue_hbm), ())
    lax.fori_loop(0, 4, make(iidx_hbm, ie_hbm), ())


_emb_call = pl.kernel(
    _emb_body,
    out_type=(jax.ShapeDtypeStruct((BSZ, EMB), jnp.float32),
              jax.ShapeDtypeStruct((BSZ, EMB), jnp.float32)),
    mesh=_MESH,
    scratch_types=[
        pltpu.VMEM((BT,), jnp.int32),
        pltpu.VMEM((BT, EMB), jnp.float32),
        pltpu.SemaphoreType.DMA,
    ],
)


def _dinv(d0_ref, d1_ref):
    deg = d0_ref[:, 0:1] + d1_ref[:, 0:1] + 1.0
    return lax.rsqrt(deg)


def _tca_body(x_ref, w_ref, d0_ref, d1_ref, o0, o1, o2, o3):
    g = jnp.dot(x_ref[...], w_ref[...],
                preferred_element_type=jnp.float32) * _dinv(d0_ref, d1_ref)
    o0[...] = g[:, 0:32]
    o1[...] = g[:, 32:64]
    o2[...] = g[:, 64:96]
    o3[...] = g[:, 96:128]


def _tca(xp, W1, deg0, deg1):
    row = lambda i: (i, 0)
    return pl.pallas_call(
        _tca_body,
        grid=(NBLK,),
        in_specs=[
            pl.BlockSpec((ROWB, EMB), row),
            pl.BlockSpec((EMB, HID), lambda i: (0, 0)),
            pl.BlockSpec((ROWB, 16), row),
            pl.BlockSpec((ROWB, 16), row),
        ],
        out_specs=[pl.BlockSpec((ROWB, 32), row)] * 4,
        out_shape=[jax.ShapeDtypeStruct((NP, 32), jnp.float32)] * 4,
    )(xp, W1, deg0, deg1)


def _tcb_body(s0, s1, s2, s3, g0, g1, g2, g3, d0, d1, w2_ref, b1_ref, o0, o1):
    dinv = _dinv(d0, d1)
    S = jnp.concatenate([s0[...], s1[...], s2[...], s3[...]], axis=1)
    G = jnp.concatenate([g0[...], g1[...], g2[...], g3[...]], axis=1)
    h1 = jnp.maximum(dinv * (S + G) + b1_ref[...], 0.0)
    g2o = jnp.dot(h1, w2_ref[...], preferred_element_type=jnp.float32) * dinv
    o0[...] = g2o[:, 0:32]
    o1[...] = g2o[:, 32:64]


def _tcb(s1s, g1s, deg0, deg1, W2, b1):
    row = lambda i: (i, 0)
    return pl.pallas_call(
        _tcb_body,
        grid=(NBLK,),
        in_specs=[pl.BlockSpec((ROWB, 32), row)] * 8 + [
            pl.BlockSpec((ROWB, 16), row),
            pl.BlockSpec((ROWB, 16), row),
            pl.BlockSpec((HID, EMB), lambda i: (0, 0)),
            pl.BlockSpec((1, HID), lambda i: (0, 0)),
        ],
        out_specs=[pl.BlockSpec((ROWB, 32), row)] * 2,
        out_shape=[jax.ShapeDtypeStruct((NP, 32), jnp.float32)] * 2,
    )(*s1s, *g1s, deg0, deg1, W2, b1.reshape(1, HID))


def _tcc_body(s0, s1, g0, g1, d0, d1, b2_ref, out):
    dinv = _dinv(d0, d1)
    S = jnp.concatenate([s0[...], s1[...]], axis=1)
    G = jnp.concatenate([g0[...], g1[...]], axis=1)
    out[...] = dinv * (S + G) + b2_ref[...]


def _tcc(s2s, g2s, deg0, deg1, b2):
    row = lambda i: (i, 0)
    return pl.pallas_call(
        _tcc_body,
        grid=(NBLK,),
        in_specs=[pl.BlockSpec((ROWB, 32), row)] * 4 + [
            pl.BlockSpec((ROWB, 16), row),
            pl.BlockSpec((ROWB, 16), row),
            pl.BlockSpec((1, EMB), lambda i: (0, 0)),
        ],
        out_specs=pl.BlockSpec((ROWB, EMB), row),
        out_shape=jax.ShapeDtypeStruct((NP, EMB), jnp.float32),
    )(*s2s, *g2s, deg0, deg1, b2.reshape(1, EMB))


def _layer_norm(z, g, b):
    mu = jnp.mean(z, axis=-1, keepdims=True)
    var = jnp.mean((z - mu) ** 2, axis=-1, keepdims=True)
    return (z - mu) * lax.rsqrt(var + 1e-5) * g + b


def _tcd_body(ue, ie, w1a, w1b, b1r, g1r, bb1, w2r, b2r, g2r, bb2, w3r, b3r,
              out):
    z = (jnp.dot(ue[...], w1a[...], preferred_element_type=jnp.float32)
         + jnp.dot(ie[...], w1b[...], preferred_element_type=jnp.float32)
         + b1r[...])
    z = jnp.maximum(_layer_norm(z, g1r[...], bb1[...]), 0.0)
    z = jnp.dot(z, w2r[...], preferred_element_type=jnp.float32) + b2r[...]
    z = jnp.maximum(_layer_norm(z, g2r[...], bb2[...]), 0.0)
    s = jnp.sum(z * w3r[...], axis=1, keepdims=True) + b3r[...]
    out[...] = 1.0 / (1.0 + jnp.exp(-s))


def _tcd(ue, ie, fc1_W, fc1_b, ln1_g, ln1_b, fc2_W, fc2_b, ln2_g, ln2_b,
         fc3_W, fc3_b):
    row = lambda i: (i, 0)
    full = lambda i: (0, 0)
    return pl.pallas_call(
        _tcd_body,
        grid=(BSZ // ROWB,),
        in_specs=[
            pl.BlockSpec((ROWB, EMB), row),
            pl.BlockSpec((ROWB, EMB), row),
            pl.BlockSpec((EMB, HID), full),
            pl.BlockSpec((EMB, HID), full),
            pl.BlockSpec((1, HID), full),
            pl.BlockSpec((1, HID), full),
            pl.BlockSpec((1, HID), full),
            pl.BlockSpec((HID, EMB), full),
            pl.BlockSpec((1, EMB), full),
            pl.BlockSpec((1, EMB), full),
            pl.BlockSpec((1, EMB), full),
            pl.BlockSpec((1, EMB), full),
            pl.BlockSpec((1, 1), full),
        ],
        out_specs=pl.BlockSpec((ROWB, 1), row),
        out_shape=jax.ShapeDtypeStruct((BSZ, 1), jnp.float32),
    )(ue, ie, fc1_W[:EMB], fc1_W[EMB:], fc1_b.reshape(1, HID),
      ln1_g.reshape(1, HID), ln1_b.reshape(1, HID),
      fc2_W, fc2_b.reshape(1, EMB), ln2_g.reshape(1, EMB),
      ln2_b.reshape(1, EMB), fc3_W.reshape(1, EMB), fc3_b.reshape(1, 1))


def kernel(user_indices, item_indices, edge_index, x, user_table, item_table,
           W1, b1, W2, b2, fc1_W, fc1_b, ln1_g, ln1_b,
           fc2_W, fc2_b, ln2_g, ln2_b, fc3_W, fc3_b):
    src = edge_index[0].astype(jnp.int32)
    dst = edge_index[1].astype(jnp.int32)
    pad = NEPAD - NE
    src = jnp.concatenate([src, jnp.zeros((pad,), jnp.int32)]).reshape(NER, BT)
    dst = jnp.concatenate([dst, jnp.full((pad,), NP - 1, jnp.int32)]
                          ).reshape(NER, BT)
    xp = jnp.pad(x, ((0, NP - NNODES), (0, 0)))
    uidx = user_indices.astype(jnp.int32) - 1
    iidx = item_indices.astype(jnp.int32) - 1 + NUSERS
    zero16 = jnp.zeros((NP, 16), jnp.float32)
    zero32 = jnp.zeros((NP, 32), jnp.float32)
    ones_rows = jnp.ones((BT, 16), jnp.float32)

    deg0, deg1 = _deg_call(dst, zero16, ones_rows)
    g1s = _tca(xp, W1, deg0, deg1)
    s10, s11 = _spmm_call(src, dst, g1s[0], g1s[1], zero32)
    s12, s13 = _spmm_call(src, dst, g1s[2], g1s[3], zero32)
    g2s = _tcb((s10, s11, s12, s13), g1s, deg0, deg1, W2, b1)
    s20, s21 = _spmm_call(src, dst, g2s[0], g2s[1], zero32)
    h2 = _tcc((s20, s21), g2s, deg0, deg1, b2)
    ue, ie = _emb_call(h2, uidx, iidx)
    out = _tcd(ue, ie, fc1_W, fc1_b, ln1_g, ln1_b,
               fc2_W, fc2_b, ln2_g, ln2_b, fc3_W, fc3_b)
    return out.reshape(BSZ)


# speedup vs baseline: 2.2304x; 2.2304x over previous
import jax
import jax.numpy as jnp
from jax import lax
from jax.experimental import pallas as pl
from jax.experimental.pallas import tpu as pltpu
from jax.experimental.pallas import tpu_sc as plsc

NUSERS = 25000
NNODES = 50000
ROWB = 256
NBLK = 196
NP = NBLK * ROWB
EMB = 64
HID = 128
NE = 800000
NC, NS = 2, 16
BT = 128
NER = 6400
NEPAD = NER * BT
BSZ = 16384
CHKD = 8
CH = 12544
ACCR = 12800
DUMP = CH
NCHUNK = NP // CH

_MESH = plsc.VectorSubcoreMesh(
    core_axis_name="c", subcore_axis_name="s", num_cores=NC, num_subcores=NS)


def _localize(idx_c, idxl, j, base):
    for gix in range(8):
        d = idx_c[j, pl.ds(gix * 16, 16)]
        dl = d - base
        ok = (dl >= 0) & (dl < CH)
        idxl[j, pl.ds(gix * 16, 16)] = jnp.where(ok, dl, DUMP)


def _zero_acc(zb, acc, s):
    z0 = s * (ACCR // NS)

    def zp(p, _):
        pltpu.sync_copy(zb, acc.at[pl.ds(z0 + p * 16, 16)])
        return ()

    lax.fori_loop(0, ACCR // NS // 16, zp, ())


def _writeback(acc, wb, out_hbm, base, s):
    w0 = s * (CH // NS)

    def wp(p, _):
        pltpu.sync_copy(acc.at[pl.ds(w0 + p * 16, 16)], wb)
        pltpu.sync_copy(wb, out_hbm.at[pl.ds(base + w0 + p * 16, 16)])
        return ()

    lax.fori_loop(0, CH // NS // 16, wp, ())


def _deg_body(dst_hbm, zero_hbm, ones_hbm, deg0_hbm, deg1_hbm,
              idx_c, idxl, ones_v, zb, wb, acc, sem):
    c = lax.axis_index("c")
    s = lax.axis_index("s")
    ndr = NER // 2 // NS
    rbase = c * (NER // 2) + s * ndr
    pltpu.sync_copy(ones_hbm, ones_v)
    pltpu.sync_copy(zero_hbm, zb)

    for chunk in range(NCHUNK):
        base = chunk * CH
        _zero_acc(zb, acc, s)
        plsc.subcore_barrier()

        def chunklet(q, _):
            pltpu.sync_copy(dst_hbm.at[pl.ds(rbase + q * CHKD, CHKD)], idx_c)
            for j in range(CHKD):
                _localize(idx_c, idxl, j, base)
                pltpu.sync_copy(ones_v, acc.at[idxl.at[j]], add=True)
            return ()

        lax.fori_loop(0, ndr // CHKD, chunklet, ())
        plsc.subcore_barrier()

        @pl.when(c == 0)
        def _():
            _writeback(acc, wb, deg0_hbm, base, s)

        @pl.when(c == 1)
        def _():
            _writeback(acc, wb, deg1_hbm, base, s)

        plsc.subcore_barrier()


_deg_call = pl.kernel(
    _deg_body,
    out_type=(jax.ShapeDtypeStruct((NP, BT), jnp.float32),
              jax.ShapeDtypeStruct((NP, BT), jnp.float32)),
    mesh=_MESH,
    scratch_types=[
        pltpu.VMEM((CHKD, BT), jnp.int32),
        pltpu.VMEM((CHKD, BT), jnp.int32),
        pltpu.VMEM((BT, BT), jnp.float32),
        pltpu.VMEM((16, BT), jnp.float32),
        pltpu.VMEM((16, BT), jnp.float32),
        pltpu.VMEM_SHARED((ACCR, BT), jnp.float32),
        pltpu.SemaphoreType.DMA,
    ],
)


def _spmm_body(src_hbm, dst_hbm, zero_hbm, g_hbm, sa_hbm, sb_hbm,
               idx_c, idxl, idxs, rows, zb, wb, acc, sem):
    c = lax.axis_index("c")
    s = lax.axis_index("s")
    ndr = NER // 2 // NS
    rbase = c * (NER // 2) + s * ndr
    pltpu.sync_copy(zero_hbm, zb)

    for chunk in range(NCHUNK):
        base = chunk * CH
        _zero_acc(zb, acc, s)
        plsc.subcore_barrier()

        def chunklet(q, _):
            pltpu.sync_copy(dst_hbm.at[pl.ds(rbase + q * CHKD, CHKD)], idx_c)
            pltpu.sync_copy(src_hbm.at[pl.ds(rbase + q * CHKD, CHKD)], idxs)
            for j in range(CHKD):
                _localize(idx_c, idxl, j, base)
                pltpu.async_copy(g_hbm.at[idxs.at[j]], rows, sem).wait()
                pltpu.sync_copy(rows, acc.at[idxl.at[j]], add=True)
            return ()

        lax.fori_loop(0, ndr // CHKD, chunklet, ())
        plsc.subcore_barrier()

        @pl.when(c == 0)
        def _():
            _writeback(acc, wb, sa_hbm, base, s)

        @pl.when(c == 1)
        def _():
            _writeback(acc, wb, sb_hbm, base, s)

        plsc.subcore_barrier()


_spmm_call = pl.kernel(
    _spmm_body,
    out_type=(jax.ShapeDtypeStruct((NP, BT), jnp.float32),
              jax.ShapeDtypeStruct((NP, BT), jnp.float32)),
    mesh=_MESH,
    scratch_types=[
        pltpu.VMEM((CHKD, BT), jnp.int32),
        pltpu.VMEM((CHKD, BT), jnp.int32),
        pltpu.VMEM((CHKD, BT), jnp.int32),
        pltpu.VMEM((BT, BT), jnp.float32),
        pltpu.VMEM((16, BT), jnp.float32),
        pltpu.VMEM((16, BT), jnp.float32),
        pltpu.VMEM_SHARED((ACCR, BT), jnp.float32),
        pltpu.SemaphoreType.DMA,
    ],
)


def _emb_body(h2_hbm, uidx_hbm, iidx_hbm, ue_hbm, ie_hbm, idx_v, rows, sem):
    c = lax.axis_index("c")
    s = lax.axis_index("s")
    wid = c * NS + s
    base = wid * (BSZ // (NC * NS))

    def make(src_idx_hbm, out_hbm):
        def body(t, _):
            off = base + t * BT
            pltpu.sync_copy(src_idx_hbm.at[pl.ds(off, BT)], idx_v)
            pltpu.async_copy(h2_hbm.at[idx_v], rows, sem).wait()
            pltpu.sync_copy(rows, out_hbm.at[pl.ds(off, BT)])
            return ()
        return body

    lax.fori_loop(0, 4, make(uidx_hbm, ue_hbm), ())
    lax.fori_loop(0, 4, make(iidx_hbm, ie_hbm), ())


_emb_call = pl.kernel(
    _emb_body,
    out_type=(jax.ShapeDtypeStruct((BSZ, HID), jnp.float32),
              jax.ShapeDtypeStruct((BSZ, HID), jnp.float32)),
    mesh=_MESH,
    scratch_types=[
        pltpu.VMEM((BT,), jnp.int32),
        pltpu.VMEM((BT, HID), jnp.float32),
        pltpu.SemaphoreType.DMA,
    ],
)


def _dinv(d0_ref, d1_ref):
    deg = d0_ref[:, 0:1] + d1_ref[:, 0:1] + 1.0
    return lax.rsqrt(deg)


def _tca_body(x_ref, w_ref, d0_ref, d1_ref, out):
    out[...] = jnp.dot(x_ref[...], w_ref[...],
                       preferred_element_type=jnp.float32) * _dinv(d0_ref, d1_ref)


def _tca(xp, W1, deg0, deg1):
    row = lambda i: (i, 0)
    return pl.pallas_call(
        _tca_body,
        grid=(NBLK,),
        in_specs=[
            pl.BlockSpec((ROWB, EMB), row),
            pl.BlockSpec((EMB, HID), lambda i: (0, 0)),
            pl.BlockSpec((ROWB, BT), row),
            pl.BlockSpec((ROWB, BT), row),
        ],
        out_specs=pl.BlockSpec((ROWB, HID), row),
        out_shape=jax.ShapeDtypeStruct((NP, HID), jnp.float32),
    )(xp, W1, deg0, deg1)


def _tcb_body(sa, sb, g1, d0, d1, w2_ref, b1_ref, out):
    dinv = _dinv(d0, d1)
    h1 = jnp.maximum(dinv * (sa[...] + sb[...] + g1[...]) + b1_ref[...], 0.0)
    g2v = jnp.dot(h1, w2_ref[...], preferred_element_type=jnp.float32) * dinv
    out[...] = jnp.concatenate([g2v, jnp.zeros_like(g2v)], axis=1)


def _tcb(s1a, s1b, g1, deg0, deg1, W2, b1):
    row = lambda i: (i, 0)
    return pl.pallas_call(
        _tcb_body,
        grid=(NBLK,),
        in_specs=[pl.BlockSpec((ROWB, BT), row)] * 5 + [
            pl.BlockSpec((HID, EMB), lambda i: (0, 0)),
            pl.BlockSpec((1, HID), lambda i: (0, 0)),
        ],
        out_specs=pl.BlockSpec((ROWB, HID), row),
        out_shape=jax.ShapeDtypeStruct((NP, HID), jnp.float32),
    )(s1a, s1b, g1, deg0, deg1, W2, b1.reshape(1, HID))


def _tcc_body(sa, sb, g2, d0, d1, b2_ref, out):
    dinv = _dinv(d0, d1)
    h2 = dinv * (sa[:, 0:EMB] + sb[:, 0:EMB] + g2[:, 0:EMB]) + b2_ref[...]
    out[...] = jnp.concatenate([h2, jnp.zeros_like(h2)], axis=1)


def _tcc(s2a, s2b, g2, deg0, deg1, b2):
    row = lambda i: (i, 0)
    return pl.pallas_call(
        _tcc_body,
        grid=(NBLK,),
        in_specs=[pl.BlockSpec((ROWB, BT), row)] * 5 + [
            pl.BlockSpec((1, EMB), lambda i: (0, 0)),
        ],
        out_specs=pl.BlockSpec((ROWB, HID), row),
        out_shape=jax.ShapeDtypeStruct((NP, HID), jnp.float32),
    )(s2a, s2b, g2, deg0, deg1, b2.reshape(1, EMB))


def _layer_norm(z, g, b):
    mu = jnp.mean(z, axis=-1, keepdims=True)
    var = jnp.mean((z - mu) ** 2, axis=-1, keepdims=True)
    return (z - mu) * lax.rsqrt(var + 1e-5) * g + b


def _tcd_body(ue, ie, w1a, w1b, b1r, g1r, bb1, w2r, b2r, g2r, bb2, w3r, b3r,
              out):
    z = (jnp.dot(ue[:, 0:EMB], w1a[...], preferred_element_type=jnp.float32)
         + jnp.dot(ie[:, 0:EMB], w1b[...], preferred_element_type=jnp.float32)
         + b1r[...])
    z = jnp.maximum(_layer_norm(z, g1r[...], bb1[...]), 0.0)
    z = jnp.dot(z, w2r[...], preferred_element_type=jnp.float32) + b2r[...]
    z = jnp.maximum(_layer_norm(z, g2r[...], bb2[...]), 0.0)
    s = jnp.sum(z * w3r[...], axis=1, keepdims=True) + b3r[...]
    out[...] = 1.0 / (1.0 + jnp.exp(-s))


def _tcd(ue, ie, fc1_W, fc1_b, ln1_g, ln1_b, fc2_W, fc2_b, ln2_g, ln2_b,
         fc3_W, fc3_b):
    row = lambda i: (i, 0)
    full = lambda i: (0, 0)
    return pl.pallas_call(
        _tcd_body,
        grid=(BSZ // ROWB,),
        in_specs=[
            pl.BlockSpec((ROWB, HID), row),
            pl.BlockSpec((ROWB, HID), row),
            pl.BlockSpec((EMB, HID), full),
            pl.BlockSpec((EMB, HID), full),
            pl.BlockSpec((1, HID), full),
            pl.BlockSpec((1, HID), full),
            pl.BlockSpec((1, HID), full),
            pl.BlockSpec((HID, EMB), full),
            pl.BlockSpec((1, EMB), full),
            pl.BlockSpec((1, EMB), full),
            pl.BlockSpec((1, EMB), full),
            pl.BlockSpec((1, EMB), full),
            pl.BlockSpec((1, 1), full),
        ],
        out_specs=pl.BlockSpec((ROWB, 1), row),
        out_shape=jax.ShapeDtypeStruct((BSZ, 1), jnp.float32),
    )(ue, ie, fc1_W[:EMB], fc1_W[EMB:], fc1_b.reshape(1, HID),
      ln1_g.reshape(1, HID), ln1_b.reshape(1, HID),
      fc2_W, fc2_b.reshape(1, EMB), ln2_g.reshape(1, EMB),
      ln2_b.reshape(1, EMB), fc3_W.reshape(1, EMB), fc3_b.reshape(1, 1))


def kernel(user_indices, item_indices, edge_index, x, user_table, item_table,
           W1, b1, W2, b2, fc1_W, fc1_b, ln1_g, ln1_b,
           fc2_W, fc2_b, ln2_g, ln2_b, fc3_W, fc3_b):
    src = edge_index[0].astype(jnp.int32)
    dst = edge_index[1].astype(jnp.int32)
    pad = NEPAD - NE
    src = jnp.concatenate([src, jnp.zeros((pad,), jnp.int32)]).reshape(NER, BT)
    dst = jnp.concatenate([dst, jnp.full((pad,), NP - 1, jnp.int32)]
                          ).reshape(NER, BT)
    xp = jnp.pad(x, ((0, NP - NNODES), (0, 0)))
    uidx = user_indices.astype(jnp.int32) - 1
    iidx = item_indices.astype(jnp.int32) - 1 + NUSERS
    zero = jnp.zeros((16, BT), jnp.float32)
    ones_rows = jnp.ones((BT, BT), jnp.float32)

    deg0, deg1 = _deg_call(dst, zero, ones_rows)
    g1 = _tca(xp, W1, deg0, deg1)
    s1a, s1b = _spmm_call(src, dst, zero, g1)
    g2 = _tcb(s1a, s1b, g1, deg0, deg1, W2, b1)
    s2a, s2b = _spmm_call(src, dst, zero, g2)
    h2 = _tcc(s2a, s2b, g2, deg0, deg1, b2)
    ue, ie = _emb_call(h2, uidx, iidx)
    out = _tcd(ue, ie, fc1_W, fc1_b, ln1_g, ln1_b,
               fc2_W, fc2_b, ln2_g, ln2_b, fc3_W, fc3_b)
    return out.reshape(BSZ)
